# baseline (speedup 1.0000x reference)
import functools

import jax
import jax.numpy as jnp
from jax import lax
from jax.experimental import pallas as pl
from jax.experimental.pallas import tpu as pltpu
from jax.experimental.pallas import tpu_sc as plsc

NC = 2
NS = 16
NW = NC * NS
LANES = 16
KQ = 4
RW = 128


def _sc_scores(cen2d, pos_rec, pos_cb, neg_rec, neg_cb,
               in_t, out_rec, *, B, K, D):
    b_per_w = B // NW
    CB = LANES
    n_chunks = b_per_w // CB
    rpc = CB * K
    NR = 64
    ndma = rpc // NR
    nrow_w = b_per_w * K // NR

    mesh = plsc.VectorSubcoreMesh(
        core_axis_name="c", subcore_axis_name="s",
        num_cores=NC, num_subcores=NS,
    )

    @functools.partial(
        pl.kernel,
        out_type=[
            jax.ShapeDtypeStruct((NW, n_chunks, LANES), jnp.float32),
            jax.ShapeDtypeStruct((NW, n_chunks, K, LANES), jnp.float32),
        ],
        mesh=mesh,
        compiler_params=pltpu.CompilerParams(
            needs_layout_passes=False, use_tc_tiling_on_sc=False),
        scratch_types=[
            pltpu.VMEM((n_chunks, CB), jnp.int32),
            pltpu.VMEM((n_chunks, CB), jnp.int32),
            pltpu.VMEM((nrow_w, NR), jnp.int32),
            pltpu.VMEM((nrow_w, NR), jnp.int32),
            pltpu.VMEM((2 * CB, RW), jnp.float32),
            pltpu.VMEM((2 * rpc, RW), jnp.float32),
            pltpu.VMEM((D, 2 * CB * 8), jnp.float32),
            pltpu.VMEM((D, LANES), jnp.float32),
            pltpu.VMEM((1, LANES), jnp.float32),
            pltpu.VMEM((K, LANES), jnp.float32),
            pltpu.VMEM((n_chunks, CB), jnp.int32),
            pltpu.SemaphoreType.DMA,
            pltpu.SemaphoreType.DMA,
        ],
    )
    def scores_kernel(cen_hbm, pr_hbm, pc_hbm, nr_hbm, ncb_hbm,
                      in_hbm, out_hbm,
                      ps_out, ns_out,
                      pidx, pcb, nidx, ncb,
                      vob, vn, vc8, vcT, pos_acc, neg_acc, cidx,
                      sem_a, sem_b):
        wid = lax.axis_index("s") * NC + lax.axis_index("c")
        iota = lax.broadcasted_iota(jnp.int32, (LANES,), 0)

        pltpu.sync_copy(pr_hbm.at[pl.ds(wid * n_chunks, n_chunks)], pidx)
        pltpu.sync_copy(pc_hbm.at[pl.ds(wid * n_chunks, n_chunks)], pcb)
        pltpu.sync_copy(nr_hbm.at[pl.ds(wid * nrow_w, nrow_w)], nidx)
        pltpu.sync_copy(ncb_hbm.at[pl.ds(wid * nrow_w, nrow_w)], ncb)
        pltpu.sync_copy(cen_hbm.at[pl.ds(wid * n_chunks, n_chunks)], cidx)

        def fire_chunk(c):
            half = jnp.bitwise_and(c, 1)
            avec = jnp.bitwise_and(cidx[c, :], jnp.int32(~7))
            for i in range(CB):
                pltpu.async_copy(
                    in_hbm.at[:, pl.ds(pl.multiple_of(avec[i], 8), 8)],
                    vc8.at[:, pl.ds((half * CB + i) * 8, 8)], sem_a)
            pltpu.async_copy(
                out_hbm.at[pidx.at[c]],
                vob.at[pl.ds(half * CB, CB)], sem_a)
            for j in range(ndma):
                pltpu.async_copy(
                    out_hbm.at[nidx.at[c * ndma + j]],
                    vn.at[pl.ds(half * rpc + j * NR, NR)], sem_b)

        fire_chunk(jnp.int32(0))

        def chunk_body(c, carry):
            half = jnp.bitwise_and(c, 1)
            nxt = c + 1

            @pl.when(nxt < n_chunks)
            def _fire():
                fire_chunk(nxt)

            pltpu.make_async_copy(
                in_hbm.at[:, pl.ds(0, CB * 8)],
                vc8.at[:, pl.ds(half * CB * 8, CB * 8)], sem_a).wait()
            pltpu.make_async_copy(
                out_hbm.at[pl.ds(0, CB)], vob.at[pl.ds(half * CB, CB)],
                sem_a).wait()
            pltpu.make_async_copy(
                out_hbm.at[pl.ds(0, rpc)], vn.at[pl.ds(half * rpc, rpc)],
                sem_b).wait()

            rows16 = half * CB + iota
            colb_p = pcb[c, :]
            crem = half * CB * 8 + iota * 8 + jnp.bitwise_and(
                cidx[c, :], jnp.int32(7))

            acc_p = jnp.zeros((LANES,), jnp.float32)
            for d in range(D):
                dcol = jnp.full((LANES,), d, jnp.int32)
                vcc = plsc.load_gather(vc8, [dcol, crem])
                voc = plsc.load_gather(vob, [rows16, colb_p + d])
                vcT[d, :] = vcc
                acc_p = acc_p + vcc * voc
            pos_acc[0, :] = acc_p

            vn_base = half * rpc
            for kq in range(K // KQ):
                rows_k = []
                cols_k = []
                for t in range(KQ):
                    k = kq * KQ + t
                    s = iota * K + k
                    flat = c * rpc + s
                    rows_k.append(vn_base + s)
                    cols_k.append(plsc.load_gather(
                        ncb, [lax.shift_right_logical(flat, 6),
                              jnp.bitwise_and(flat, NR - 1)]))
                accs = [jnp.zeros((LANES,), jnp.float32)] * KQ
                for d in range(D):
                    cv = vcT[d, :]
                    for t in range(KQ):
                        vnc = plsc.load_gather(vn, [rows_k[t], cols_k[t] + d])
                        accs[t] = accs[t] + cv * vnc
                for t in range(KQ):
                    neg_acc[kq * KQ + t, :] = accs[t]

            pltpu.sync_copy(pos_acc, ps_out.at[wid, pl.ds(c, 1)])
            pltpu.sync_copy(neg_acc, ns_out.at[wid, c])
            return carry

        lax.fori_loop(0, n_chunks, chunk_body, 0)

    return scores_kernel(cen2d, pos_rec, pos_cb, neg_rec, neg_cb,
                         in_t, out_rec)


def _tc_loss(ps2d, ns2d, *, B):
    inv_b = 1.0 / float(B)

    def body(ps_ref, ns_ref, o_ref):
        def log_sig(x):
            return jnp.minimum(x, 0.0) - jnp.log(1.0 + jnp.exp(-jnp.abs(x)))

        pos_l = jnp.sum(log_sig(ps_ref[...]))
        neg_l = jnp.sum(log_sig(-ns_ref[...]))
        o_ref[...] = jnp.reshape(-(pos_l + neg_l) * inv_b, (1, 1))

    out = pl.pallas_call(
        body,
        out_shape=jax.ShapeDtypeStruct((1, 1), jnp.float32),
    )(ps2d, ns2d)
    return out[0, 0]


def kernel(centers, pos_contexts, neg_contexts, in_table, out_table):
    B = centers.shape[0]
    K = neg_contexts.shape[1]
    D = in_table.shape[1]
    rpr = RW // D

    def split(a, w):
        a = a.astype(jnp.int32)
        return (a // rpr).reshape(-1, w), ((a % rpr) * D).reshape(-1, w)

    cen2d = centers.astype(jnp.int32).reshape(-1, LANES)
    pos_rec, pos_cb = split(pos_contexts, LANES)
    neg_rec, neg_cb = split(neg_contexts, 64)

    in_t = jnp.transpose(in_table)
    out_rec = out_table.reshape(-1, RW)

    ps, ns = _sc_scores(cen2d, pos_rec, pos_cb, neg_rec, neg_cb,
                        in_t, out_rec, B=B, K=K, D=D)
    return _tc_loss(ps.reshape(-1, 128), ns.reshape(-1, 128), B=B)


# speedup vs baseline: 2.9866x; 2.9866x over previous
import functools

import jax
import jax.numpy as jnp
from jax import lax
from jax.experimental import pallas as pl
from jax.experimental.pallas import tpu as pltpu
from jax.experimental.pallas import tpu_sc as plsc

NC = 2
NS = 16
NW = NC * NS
LANES = 16
KQ = 4


def _sc_scores(cen2d, pos2d, neg2d, in_table, out_table, *, B, K, D):
    b_per_w = B // NW
    n_ib = b_per_w // 128
    n_nb = b_per_w * K // 128
    CB = 64
    n_chunks = b_per_w // CB
    rows_per_chunk = CB * K
    ndma = rows_per_chunk // 128
    ngrp = CB // LANES

    mesh = plsc.VectorSubcoreMesh(
        core_axis_name="c", subcore_axis_name="s",
        num_cores=NC, num_subcores=NS,
    )

    @functools.partial(
        pl.kernel,
        out_type=[
            jax.ShapeDtypeStruct((NW, n_chunks, ngrp, LANES), jnp.float32),
            jax.ShapeDtypeStruct((NW, n_chunks, K, ngrp, LANES), jnp.float32),
        ],
        mesh=mesh,
        compiler_params=pltpu.CompilerParams(
            needs_layout_passes=False, use_tc_tiling_on_sc=False),
        scratch_types=[
            pltpu.VMEM((n_ib, 128), jnp.int32),
            pltpu.VMEM((n_ib, 128), jnp.int32),
            pltpu.VMEM((n_nb, 128), jnp.int32),
            pltpu.VMEM((2 * CB, D), jnp.float32),
            pltpu.VMEM((2 * CB, D), jnp.float32),
            pltpu.VMEM((2 * rows_per_chunk, D), jnp.float32),
            pltpu.VMEM((D, ngrp, LANES), jnp.float32),
            pltpu.VMEM((ngrp, LANES), jnp.float32),
            pltpu.VMEM((K, ngrp, LANES), jnp.float32),
            pltpu.SemaphoreType.DMA,
            pltpu.SemaphoreType.DMA,
        ],
    )
    def scores_kernel(cen_hbm, pos_hbm, neg_hbm, in_hbm, out_hbm,
                      ps_out, ns_out,
                      cidx, pidx, nidx, vcb, vob, vn, vcT, pos_acc, neg_acc,
                      sem_a, sem_b):
        wid = lax.axis_index("s") * NC + lax.axis_index("c")
        iota = lax.broadcasted_iota(jnp.int32, (LANES,), 0)

        pltpu.sync_copy(cen_hbm.at[pl.ds(wid * n_ib, n_ib)], cidx)
        pltpu.sync_copy(pos_hbm.at[pl.ds(wid * n_ib, n_ib)], pidx)
        pltpu.sync_copy(neg_hbm.at[pl.ds(wid * n_nb, n_nb)], nidx)

        def fire_chunk(c):
            half = jnp.bitwise_and(c, 1)
            crow = lax.div(c, 2)
            coff = jnp.bitwise_and(c, 1) * CB
            pltpu.async_copy(
                in_hbm.at[cidx.at[crow, pl.ds(coff, CB)]],
                vcb.at[pl.ds(half * CB, CB)], sem_a)
            pltpu.async_copy(
                out_hbm.at[pidx.at[crow, pl.ds(coff, CB)]],
                vob.at[pl.ds(half * CB, CB)], sem_a)
            for j in range(ndma):
                pltpu.async_copy(
                    out_hbm.at[nidx.at[c * ndma + j]],
                    vn.at[pl.ds(half * rows_per_chunk + j * 128, 128)], sem_b)

        fire_chunk(jnp.int32(0))

        def chunk_body(c, carry):
            half = jnp.bitwise_and(c, 1)
            nxt = c + 1

            @pl.when(nxt < n_chunks)
            def _fire():
                fire_chunk(nxt)

            pltpu.make_async_copy(
                in_hbm.at[pl.ds(0, CB)], vcb.at[pl.ds(half * CB, CB)],
                sem_a).wait()
            pltpu.make_async_copy(
                in_hbm.at[pl.ds(0, CB)], vob.at[pl.ds(half * CB, CB)],
                sem_a).wait()
            pltpu.make_async_copy(
                out_hbm.at[pl.ds(0, rows_per_chunk)],
                vn.at[pl.ds(half * rows_per_chunk, rows_per_chunk)],
                sem_b).wait()

            vn_base = half * rows_per_chunk

            def g_body(g, carry2):
                rows16 = half * CB + g * LANES + iota
                acc_p = jnp.zeros((LANES,), jnp.float32)
                for d in range(D):
                    col = jnp.full((LANES,), d, jnp.int32)
                    vcc = plsc.load_gather(vcb, [rows16, col])
                    voc = plsc.load_gather(vob, [rows16, col])
                    vcT[d, g, :] = vcc
                    acc_p = acc_p + vcc * voc
                pos_acc[g, :] = acc_p

                rows_n0 = vn_base + (g * LANES + iota) * K
                for kq in range(K // KQ):
                    rows_k = [rows_n0 + (kq * KQ + t) for t in range(KQ)]
                    accs = [jnp.zeros((LANES,), jnp.float32)] * KQ
                    for d in range(D):
                        col = jnp.full((LANES,), d, jnp.int32)
                        cv = vcT[d, g, :]
                        for t in range(KQ):
                            vnc = plsc.load_gather(vn, [rows_k[t], col])
                            accs[t] = accs[t] + cv * vnc
                    for t in range(KQ):
                        neg_acc[kq * KQ + t, g, :] = accs[t]
                return carry2

            lax.fori_loop(0, ngrp, g_body, 0)
            pltpu.sync_copy(pos_acc, ps_out.at[wid, c])
            pltpu.sync_copy(neg_acc, ns_out.at[wid, c])
            return carry

        lax.fori_loop(0, n_chunks, chunk_body, 0)

    return scores_kernel(cen2d, pos2d, neg2d, in_table, out_table)


def _tc_loss(ps2d, ns2d, *, B):
    inv_b = 1.0 / float(B)

    def body(ps_ref, ns_ref, o_ref):
        def log_sig(x):
            return jnp.minimum(x, 0.0) - jnp.log(1.0 + jnp.exp(-jnp.abs(x)))

        pos_l = jnp.sum(log_sig(ps_ref[...]))
        neg_l = jnp.sum(log_sig(-ns_ref[...]))
        o_ref[...] = jnp.reshape(-(pos_l + neg_l) * inv_b, (1, 1))

    out = pl.pallas_call(
        body,
        out_shape=jax.ShapeDtypeStruct((1, 1), jnp.float32),
    )(ps2d, ns2d)
    return out[0, 0]


def kernel(centers, pos_contexts, neg_contexts, in_table, out_table):
    B = centers.shape[0]
    K = neg_contexts.shape[1]
    D = in_table.shape[1]

    cen2d = centers.astype(jnp.int32).reshape(-1, 128)
    pos2d = pos_contexts.astype(jnp.int32).reshape(-1, 128)
    neg2d = neg_contexts.astype(jnp.int32).reshape(-1, 128)

    ps, ns = _sc_scores(cen2d, pos2d, neg2d, in_table, out_table,
                        B=B, K=K, D=D)
    return _tc_loss(ps.reshape(-1, 128), ns.reshape(-1, 128), B=B)


# speedup vs baseline: 2.9887x; 1.0007x over previous
import functools

import jax
import jax.numpy as jnp
from jax import lax
from jax.experimental import pallas as pl
from jax.experimental.pallas import tpu as pltpu
from jax.experimental.pallas import tpu_sc as plsc

NC = 2
NS = 16
NW = NC * NS
LANES = 16
KQ = 4


def _sc_scores(cen2d, pos2d, neg2d, in_table, out_table, *, B, K, D):
    b_per_w = B // NW
    n_ib = b_per_w // 128
    n_nb = b_per_w * K // 128
    CB = 64
    n_chunks = b_per_w // CB
    rows_per_chunk = CB * K
    ndma = rows_per_chunk // 128
    ngrp = CB // LANES

    mesh = plsc.VectorSubcoreMesh(
        core_axis_name="c", subcore_axis_name="s",
        num_cores=NC, num_subcores=NS,
    )

    @functools.partial(
        pl.kernel,
        out_type=[
            jax.ShapeDtypeStruct((NW, n_chunks, ngrp, LANES), jnp.float32),
            jax.ShapeDtypeStruct((NW, n_chunks, K, ngrp, LANES), jnp.float32),
        ],
        mesh=mesh,
        compiler_params=pltpu.CompilerParams(
            needs_layout_passes=False, use_tc_tiling_on_sc=False),
        scratch_types=[
            pltpu.VMEM((n_ib, 128), jnp.int32),
            pltpu.VMEM((n_ib, 128), jnp.int32),
            pltpu.VMEM((n_nb, 128), jnp.int32),
            pltpu.VMEM((2 * CB, D), jnp.float32),
            pltpu.VMEM((2 * CB, D), jnp.float32),
            pltpu.VMEM((2 * rows_per_chunk, D), jnp.float32),
            pltpu.VMEM((D, ngrp, LANES), jnp.float32),
            pltpu.VMEM((ngrp, LANES), jnp.float32),
            pltpu.VMEM((K, ngrp, LANES), jnp.float32),
            pltpu.SemaphoreType.DMA,
            pltpu.SemaphoreType.DMA,
            pltpu.SemaphoreType.DMA,
        ],
    )
    def scores_kernel(cen_hbm, pos_hbm, neg_hbm, in_hbm, out_hbm,
                      ps_out, ns_out,
                      cidx, pidx, nidx, vcb, vob, vn, vcT, pos_acc, neg_acc,
                      sem_a, sem_b, sem_c):
        wid = lax.axis_index("s") * NC + lax.axis_index("c")
        iota = lax.broadcasted_iota(jnp.int32, (LANES,), 0)

        pltpu.sync_copy(cen_hbm.at[pl.ds(wid * n_ib, n_ib)], cidx)
        pltpu.sync_copy(pos_hbm.at[pl.ds(wid * n_ib, n_ib)], pidx)
        pltpu.sync_copy(neg_hbm.at[pl.ds(wid * n_nb, n_nb)], nidx)

        def fire_chunk(c):
            half = jnp.bitwise_and(c, 1)
            crow = lax.div(c, 2)
            coff = jnp.bitwise_and(c, 1) * CB
            pltpu.async_copy(
                in_hbm.at[cidx.at[crow, pl.ds(coff, CB)]],
                vcb.at[pl.ds(half * CB, CB)], sem_a)
            pltpu.async_copy(
                out_hbm.at[pidx.at[crow, pl.ds(coff, CB)]],
                vob.at[pl.ds(half * CB, CB)], sem_a)
            for j in range(ndma):
                pltpu.async_copy(
                    out_hbm.at[nidx.at[c * ndma + j]],
                    vn.at[pl.ds(half * rows_per_chunk + j * 128, 128)], sem_b)

        fire_chunk(jnp.int32(0))

        def chunk_body(c, carry):
            half = jnp.bitwise_and(c, 1)
            nxt = c + 1

            @pl.when(nxt < n_chunks)
            def _fire():
                fire_chunk(nxt)

            pltpu.make_async_copy(
                in_hbm.at[pl.ds(0, CB)], vcb.at[pl.ds(half * CB, CB)],
                sem_a).wait()
            pltpu.make_async_copy(
                in_hbm.at[pl.ds(0, CB)], vob.at[pl.ds(half * CB, CB)],
                sem_a).wait()
            pltpu.make_async_copy(
                out_hbm.at[pl.ds(0, rows_per_chunk)],
                vn.at[pl.ds(half * rows_per_chunk, rows_per_chunk)],
                sem_b).wait()

            @pl.when(c > 0)
            def _drain_scores():
                pltpu.make_async_copy(
                    ps_out.at[wid, 0], pos_acc, sem_c).wait()
                pltpu.make_async_copy(
                    ns_out.at[wid, 0], neg_acc, sem_c).wait()

            vn_base = half * rows_per_chunk

            def g_body(g, carry2):
                rows16 = half * CB + g * LANES + iota
                acc_p = jnp.zeros((LANES,), jnp.float32)
                for d in range(D):
                    col = jnp.full((LANES,), d, jnp.int32)
                    vcc = plsc.load_gather(vcb, [rows16, col])
                    voc = plsc.load_gather(vob, [rows16, col])
                    vcT[d, g, :] = vcc
                    acc_p = acc_p + vcc * voc
                pos_acc[g, :] = acc_p

                rows_n0 = vn_base + (g * LANES + iota) * K
                for kq in range(K // KQ):
                    rows_k = [rows_n0 + (kq * KQ + t) for t in range(KQ)]
                    accs = [jnp.zeros((LANES,), jnp.float32)] * KQ
                    for d in range(D):
                        col = jnp.full((LANES,), d, jnp.int32)
                        cv = vcT[d, g, :]
                        for t in range(KQ):
                            vnc = plsc.load_gather(vn, [rows_k[t], col])
                            accs[t] = accs[t] + cv * vnc
                    for t in range(KQ):
                        neg_acc[kq * KQ + t, g, :] = accs[t]
                return carry2

            lax.fori_loop(0, ngrp, g_body, 0)
            pltpu.async_copy(pos_acc, ps_out.at[wid, c], sem_c)
            pltpu.async_copy(neg_acc, ns_out.at[wid, c], sem_c)
            return carry

        lax.fori_loop(0, n_chunks, chunk_body, 0)
        pltpu.make_async_copy(ps_out.at[wid, 0], pos_acc, sem_c).wait()
        pltpu.make_async_copy(ns_out.at[wid, 0], neg_acc, sem_c).wait()

    return scores_kernel(cen2d, pos2d, neg2d, in_table, out_table)


def _tc_loss(ps2d, ns2d, *, B):
    inv_b = 1.0 / float(B)

    def body(ps_ref, ns_ref, o_ref):
        def log_sig(x):
            return jnp.minimum(x, 0.0) - jnp.log(1.0 + jnp.exp(-jnp.abs(x)))

        pos_l = jnp.sum(log_sig(ps_ref[...]))
        neg_l = jnp.sum(log_sig(-ns_ref[...]))
        o_ref[...] = jnp.reshape(-(pos_l + neg_l) * inv_b, (1, 1))

    out = pl.pallas_call(
        body,
        out_shape=jax.ShapeDtypeStruct((1, 1), jnp.float32),
    )(ps2d, ns2d)
    return out[0, 0]


def kernel(centers, pos_contexts, neg_contexts, in_table, out_table):
    B = centers.shape[0]
    K = neg_contexts.shape[1]
    D = in_table.shape[1]

    cen2d = centers.astype(jnp.int32).reshape(-1, 128)
    pos2d = pos_contexts.astype(jnp.int32).reshape(-1, 128)
    neg2d = neg_contexts.astype(jnp.int32).reshape(-1, 128)

    ps, ns = _sc_scores(cen2d, pos2d, neg2d, in_table, out_table,
                        B=B, K=K, D=D)
    return _tc_loss(ps.reshape(-1, 128), ns.reshape(-1, 128), B=B)
